# Initial kernel scaffold; baseline (speedup 1.0000x reference)
import jax
import jax.numpy as jnp
from jax.experimental import pallas as pl


def kernel(inputs, embedding):
    raise NotImplementedError("write your pallas kernel here")


# speedup vs baseline: 1.3684x; 1.3684x over previous
import jax
import jax.numpy as jnp
from jax.experimental import pallas as pl

NUM_EMB = 512
EMB_DIM = 64
PIX_BLOCK = 256


def _vq_block(xp_ref, embT_ref, emb_ref, out_st_ref, out_q_ref):
    x = xp_ref[...]
    acc = jnp.zeros((x.shape[0], NUM_EMB), jnp.float32)
    for c in range(EMB_DIM):
        t = x[:, c:c + 1] - embT_ref[c:c + 1, :]
        acc = acc + t * t
    d = jnp.sqrt(acc)
    m = jnp.min(d, axis=1, keepdims=True)
    lane = jax.lax.broadcasted_iota(jnp.int32, (x.shape[0], NUM_EMB), 1)
    idx = jnp.min(jnp.where(d == m, lane, jnp.int32(NUM_EMB)), axis=1,
                  keepdims=True)
    onehot = (lane == idx).astype(jnp.float32)
    q = jnp.dot(onehot, emb_ref[...], precision=jax.lax.Precision.HIGHEST,
                preferred_element_type=jnp.float32)
    out_q_ref[...] = q
    out_st_ref[...] = x + (q - x)


def kernel(inputs, embedding):
    B, C, H, W = inputs.shape
    P = B * H * W
    xp = inputs.transpose(0, 2, 3, 1).reshape(P, C)
    embT = embedding.T
    grid = (P // PIX_BLOCK,)
    out_st, out_q = pl.pallas_call(
        _vq_block,
        grid=grid,
        in_specs=[
            pl.BlockSpec((PIX_BLOCK, C), lambda i: (i, 0)),
            pl.BlockSpec((C, NUM_EMB), lambda i: (0, 0)),
            pl.BlockSpec((NUM_EMB, C), lambda i: (0, 0)),
        ],
        out_specs=[
            pl.BlockSpec((PIX_BLOCK, C), lambda i: (i, 0)),
            pl.BlockSpec((PIX_BLOCK, C), lambda i: (i, 0)),
        ],
        out_shape=[
            jax.ShapeDtypeStruct((P, C), jnp.float32),
            jax.ShapeDtypeStruct((P, C), jnp.float32),
        ],
    )(xp, embT, embedding)
    out_st = out_st.reshape(B, H, W, C).transpose(0, 3, 1, 2)
    out_q = out_q.reshape(B, H, W, C).transpose(0, 3, 1, 2)
    return (out_st, out_q)


# speedup vs baseline: 2.2412x; 1.6378x over previous
import jax
import jax.numpy as jnp
from jax.experimental import pallas as pl

NUM_EMB = 512
EMB_DIM = 64
TOPK = 4


def _vq_kernel(xT_ref, embT_ref, emb_ref, out_stT_ref, out_qT_ref):
    xT = xT_ref[...]
    embT = embT_ref[...]
    emb = emb_ref[...]
    P = xT.shape[1]

    sT = jnp.dot(emb, xT, preferred_element_type=jnp.float32)
    en = jnp.sum(emb * emb, axis=1, keepdims=True)
    d2m = en - 2.0 * sT

    sub = jax.lax.broadcasted_iota(jnp.int32, (NUM_EMB, P), 0)
    cand_idx = []
    work = d2m
    for _ in range(TOPK):
        m = jnp.min(work, axis=0, keepdims=True)
        i = jnp.min(jnp.where(work == m, sub, jnp.int32(NUM_EMB)),
                    axis=0, keepdims=True)
        cand_idx.append(i)
        work = jnp.where(sub == i, jnp.float32(jnp.inf), work)

    best_d = None
    for k in range(TOPK):
        ohT = (sub == cand_idx[k]).astype(jnp.float32)
        gT = jnp.dot(embT, ohT, precision=jax.lax.Precision.HIGHEST,
                     preferred_element_type=jnp.float32)
        acc = jnp.zeros((1, P), jnp.float32)
        for c in range(EMB_DIM):
            t = xT[c:c + 1, :] - gT[c:c + 1, :]
            acc = acc + t * t
        d = jnp.sqrt(acc)
        if best_d is None:
            best_d, best_i, best_g = d, cand_idx[k], gT
        else:
            better = (d < best_d) | ((d == best_d) & (cand_idx[k] < best_i))
            best_d = jnp.where(better, d, best_d)
            best_i = jnp.where(better, cand_idx[k], best_i)
            best_g = jnp.where(jnp.broadcast_to(better, gT.shape), gT, best_g)

    out_qT_ref[...] = best_g
    out_stT_ref[...] = xT + (best_g - xT)


def kernel(inputs, embedding):
    B, C, H, W = inputs.shape
    P = B * H * W
    xT = inputs.transpose(1, 0, 2, 3).reshape(C, P)
    embT = embedding.T
    out_stT, out_qT = pl.pallas_call(
        _vq_kernel,
        in_specs=[
            pl.BlockSpec((C, P), lambda: (0, 0)),
            pl.BlockSpec((C, NUM_EMB), lambda: (0, 0)),
            pl.BlockSpec((NUM_EMB, C), lambda: (0, 0)),
        ],
        out_specs=[
            pl.BlockSpec((C, P), lambda: (0, 0)),
            pl.BlockSpec((C, P), lambda: (0, 0)),
        ],
        out_shape=[
            jax.ShapeDtypeStruct((C, P), jnp.float32),
            jax.ShapeDtypeStruct((C, P), jnp.float32),
        ],
    )(xT, embT, embedding)
    out_st = out_stT.reshape(C, B, H, W).transpose(1, 0, 2, 3)
    out_q = out_qT.reshape(C, B, H, W).transpose(1, 0, 2, 3)
    return (out_st, out_q)


# speedup vs baseline: 2.3417x; 1.0449x over previous
import jax
import jax.numpy as jnp
from jax.experimental import pallas as pl

NUM_EMB = 512
EMB_DIM = 64
TOPK = 4


def _vq_kernel(xT_ref, embT_ref, emb_ref, out_stT_ref, out_qT_ref):
    xT = xT_ref[...]
    embT = embT_ref[...]
    emb = emb_ref[...]
    P = xT.shape[1]

    sT = jnp.dot(emb, xT, preferred_element_type=jnp.float32)
    en = jnp.sum(emb * emb, axis=1, keepdims=True)
    d2m = en - 2.0 * sT

    kb = jax.lax.bitcast_convert_type(d2m, jnp.int32)
    key = kb ^ jax.lax.shift_right_logical(
        jax.lax.shift_right_arithmetic(kb, 31), 1)
    sub = jax.lax.broadcasted_iota(jnp.int32, (NUM_EMB, P), 0)
    work = (key & jnp.int32(~511)) | sub

    best_d = None
    for k in range(TOPK):
        m = jnp.min(work, axis=0, keepdims=True)
        eq = work == m
        ohT = jnp.where(eq, jnp.float32(1.0), jnp.float32(0.0))
        work = jnp.where(eq, jnp.int32(0x7FFFFFFF), work)
        cand_i = m & jnp.int32(511)
        gT = jnp.dot(embT, ohT, precision=jax.lax.Precision.HIGHEST,
                     preferred_element_type=jnp.float32)
        acc = jnp.zeros((1, P), jnp.float32)
        for c in range(EMB_DIM):
            t = xT[c:c + 1, :] - gT[c:c + 1, :]
            acc = acc + t * t
        d = jnp.sqrt(acc)
        if best_d is None:
            best_d, best_i, best_g = d, cand_i, gT
        else:
            better = (d < best_d) | ((d == best_d) & (cand_i < best_i))
            best_d = jnp.where(better, d, best_d)
            best_i = jnp.where(better, cand_i, best_i)
            best_g = jnp.where(jnp.broadcast_to(better, gT.shape), gT, best_g)

    out_qT_ref[...] = best_g
    out_stT_ref[...] = xT + (best_g - xT)


def kernel(inputs, embedding):
    B, C, H, W = inputs.shape
    P = B * H * W
    xT = inputs.transpose(1, 0, 2, 3).reshape(C, P)
    embT = embedding.T
    out_stT, out_qT = pl.pallas_call(
        _vq_kernel,
        in_specs=[
            pl.BlockSpec((C, P), lambda: (0, 0)),
            pl.BlockSpec((C, NUM_EMB), lambda: (0, 0)),
            pl.BlockSpec((NUM_EMB, C), lambda: (0, 0)),
        ],
        out_specs=[
            pl.BlockSpec((C, P), lambda: (0, 0)),
            pl.BlockSpec((C, P), lambda: (0, 0)),
        ],
        out_shape=[
            jax.ShapeDtypeStruct((C, P), jnp.float32),
            jax.ShapeDtypeStruct((C, P), jnp.float32),
        ],
    )(xT, embT, embedding)
    out_st = out_stT.reshape(C, B, H, W).transpose(1, 0, 2, 3)
    out_q = out_qT.reshape(C, B, H, W).transpose(1, 0, 2, 3)
    return (out_st, out_q)
